# Initial kernel scaffold; baseline (speedup 1.0000x reference)
import jax
import jax.numpy as jnp
from jax.experimental import pallas as pl


def kernel(x):
    raise NotImplementedError("write your pallas kernel here")


# speedup vs baseline: 4.2206x; 4.2206x over previous
import functools

import jax
import jax.numpy as jnp
from jax.experimental import pallas as pl
from jax.experimental.pallas import tpu as pltpu

_TRIM = 32
_N = 256
_EPS = 1e-12
_W = 1024


def _ce_large(x, j, k, w):
    n = x.shape[0]
    b = n // (2 * j)
    y = x.reshape(b, 2, j, w)
    a = y[:, 0]
    c = y[:, 1]
    mn = jnp.minimum(a, c)
    mx = jnp.maximum(a, c)
    blk = jax.lax.broadcasted_iota(jnp.int32, (b, 1, 1), 0)
    asc = ((blk * (2 * j)) & k) == 0
    lo = jnp.where(asc, mn, mx)
    hi = jnp.where(asc, mx, mn)
    return jnp.concatenate([lo[:, None], hi[:, None]], axis=1).reshape(n, w)


def _ce_small(x, j, k):
    n = x.shape[0]
    row = jax.lax.broadcasted_iota(jnp.int32, (n, 1), 0)
    upper = (row & j) != 0
    down = pltpu.roll(x, n - j, axis=0)
    up = pltpu.roll(x, j, axis=0)
    partner = jnp.where(upper, up, down)
    asc = (row & k) == 0
    take_min = upper != asc
    return jnp.where(take_min,
                     jnp.minimum(x, partner),
                     jnp.maximum(x, partner))


def _trim_body(x_ref, o_ref):
    x = x_ref[...]
    n = x.shape[0]
    w = x.shape[1]
    k = 2
    while k <= n:
        j = k // 2
        while j >= 1:
            if j >= 8:
                x = _ce_large(x, j, k, w)
            else:
                x = _ce_small(x, j, k)
            j //= 2
        k *= 2
    row = jax.lax.broadcasted_iota(jnp.int32, (n, 1), 0)
    keep = (row >= _TRIM) & (row < n - _TRIM)
    s = jnp.sum(jnp.where(keep, x, 0.0), axis=0, keepdims=True)
    u = s / (n - 2 * _TRIM)
    o_ref[...] = jnp.log(jnp.maximum(u, _EPS))


@jax.jit
def kernel(x):
    n, d = x.shape
    grid = pl.cdiv(d, _W)
    out = pl.pallas_call(
        _trim_body,
        grid=(grid,),
        in_specs=[pl.BlockSpec((n, _W), lambda i: (0, i))],
        out_specs=pl.BlockSpec((1, _W), lambda i: (0, i)),
        out_shape=jax.ShapeDtypeStruct((1, d), x.dtype),
        compiler_params=pltpu.CompilerParams(
            dimension_semantics=("parallel",),
        ),
    )(x)
    return out.reshape(d)


# speedup vs baseline: 5.4222x; 1.2847x over previous
import numpy as np

import jax
import jax.numpy as jnp
from jax.experimental import pallas as pl
from jax.experimental.pallas import tpu as pltpu

_TRIM = 32
_N = 256
_EPS = 1e-12
_W = 1024

def _logical(p):
    return ((p >> 3) | (p << 5)) & (_N - 1)


def _ce_reshape(x, jp, k, w):
    n = x.shape[0]
    bb = n // (2 * jp)
    y = x.reshape(bb, 2, jp, w)
    a = y[:, 0]
    c = y[:, 1]
    mn = jnp.minimum(a, c)
    mx = jnp.maximum(a, c)
    blk = jax.lax.broadcasted_iota(jnp.int32, (bb, jp, 1), 0)
    sub = jax.lax.broadcasted_iota(jnp.int32, (bb, jp, 1), 1)
    asc = (_logical(blk * (2 * jp) + sub) & k) == 0
    lo = jnp.where(asc, mn, mx)
    hi = jnp.where(asc, mx, mn)
    return jnp.concatenate([lo[:, None], hi[:, None]], axis=1).reshape(n, w)


def _ce_roll(x, jp, k):
    n = x.shape[0]
    p = jax.lax.broadcasted_iota(jnp.int32, (n, 1), 0)
    upper = (p & jp) != 0
    down = pltpu.roll(x, n - jp, axis=0)
    up = pltpu.roll(x, jp, axis=0)
    partner = jnp.where(upper, up, down)
    asc = (_logical(p) & k) == 0
    take_min = upper != asc
    return jnp.where(take_min, jnp.minimum(x, partner), jnp.maximum(x, partner))


def _trim_body(x_ref, o_ref):
    x = x_ref[...]
    n = x.shape[0]
    w = x.shape[1]
    k = 2
    while k <= n:
        b = k.bit_length() - 2
        while b >= 0:
            jp = 1 << ((b + 3) % 8)
            if jp >= 8:
                x = _ce_reshape(x, jp, k, w)
            else:
                x = _ce_roll(x, jp, k)
            b -= 1
        k *= 2
    p = jax.lax.broadcasted_iota(jnp.int32, (n, 1), 0)
    l = _logical(p)
    keep = (l >= _TRIM) & (l < _N - _TRIM)
    s = jnp.sum(jnp.where(keep, x, 0.0), axis=0, keepdims=True)
    u = s / (n - 2 * _TRIM)
    o_ref[...] = jnp.log(jnp.maximum(u, _EPS))


@jax.jit
def kernel(x):
    n, d = x.shape
    grid = pl.cdiv(d, _W)
    out = pl.pallas_call(
        _trim_body,
        grid=(grid,),
        in_specs=[pl.BlockSpec((n, _W), lambda i: (0, i))],
        out_specs=pl.BlockSpec((1, _W), lambda i: (0, i)),
        out_shape=jax.ShapeDtypeStruct((1, d), x.dtype),
        compiler_params=pltpu.CompilerParams(
            dimension_semantics=("parallel",),
        ),
    )(x)
    return out.reshape(d)


# speedup vs baseline: 8.4534x; 1.5590x over previous
import jax
import jax.numpy as jnp
from jax.experimental import pallas as pl
from jax.experimental.pallas import tpu as pltpu

_TRIM = 32
_N = 256
_EPS = 1e-12
_W = 128
_NV = _N // 8


def _trim_body(x_ref, o_ref):
    v = [x_ref[i * 8:(i + 1) * 8, :] for i in range(_NV)]
    w = v[0].shape[1]

    s = jax.lax.broadcasted_iota(jnp.int32, (8, 1), 0)
    asc_s = {32: (s & 1) == 0, 64: (s & 2) == 0, 128: (s & 4) == 0}
    upper_t = {1: (s & 1) != 0, 2: (s & 2) != 0, 4: (s & 4) != 0}

    def cross(jv, k):
        kb = k.bit_length() - 1
        for a in range(_NV):
            if a & jv:
                continue
            b = a ^ jv
            mn = jnp.minimum(v[a], v[b])
            mx = jnp.maximum(v[a], v[b])
            if kb >= 8:
                v[a], v[b] = mn, mx
            elif kb <= 4:
                if (a & k) == 0:
                    v[a], v[b] = mn, mx
                else:
                    v[a], v[b] = mx, mn
            else:
                m = asc_s[k]
                v[a] = jnp.where(m, mn, mx)
                v[b] = jnp.where(m, mx, mn)

    def sublane(t, k):
        kb = k.bit_length() - 1
        up = upper_t[t]
        if kb >= 8:
            tm = jnp.logical_not(up)
        else:
            tm = up != asc_s[k]
        for a in range(_NV):
            y = v[a]
            if t == 4:
                partner = pltpu.roll(y, 4, axis=0)
            else:
                partner = jnp.where(up,
                                    pltpu.roll(y, t, axis=0),
                                    pltpu.roll(y, 8 - t, axis=0))
            v[a] = jnp.where(tm, jnp.minimum(y, partner),
                             jnp.maximum(y, partner))

    k = 2
    while k <= _N:
        j = k // 2
        while j >= 1:
            b = j.bit_length() - 1
            if b <= 4:
                cross(1 << b, k)
            else:
                sublane(1 << (b - 5), k)
            j //= 2
        k *= 2

    tot = v[0]
    for a in range(1, _NV):
        tot = tot + v[a]
    keep = (s >= 1) & (s <= 6)
    tsum = jnp.sum(jnp.where(keep, tot, 0.0), axis=0, keepdims=True)
    u = tsum / (_N - 2 * _TRIM)
    o_ref[...] = jnp.log(jnp.maximum(u, _EPS))


@jax.jit
def kernel(x):
    n, d = x.shape
    grid = pl.cdiv(d, _W)
    out = pl.pallas_call(
        _trim_body,
        grid=(grid,),
        in_specs=[pl.BlockSpec((n, _W), lambda i: (0, i))],
        out_specs=pl.BlockSpec((1, _W), lambda i: (0, i)),
        out_shape=jax.ShapeDtypeStruct((1, d), x.dtype),
        compiler_params=pltpu.CompilerParams(
            dimension_semantics=("parallel",),
        ),
    )(x)
    return out.reshape(d)


# speedup vs baseline: 13.0434x; 1.5430x over previous
import jax
import jax.numpy as jnp
from jax.experimental import pallas as pl
from jax.experimental.pallas import tpu as pltpu

_TRIM = 32
_N = 256
_EPS = 1e-12
_W = 128
_NV = _N // 8


def _chunk(x_ref, o_ref, c):
    base = c * _W
    v = [x_ref[i * 8:(i + 1) * 8, pl.ds(base, _W)] for i in range(_NV)]

    s = jax.lax.broadcasted_iota(jnp.int32, (8, 1), 0)
    asc_s = {32: (s & 1) == 0, 64: (s & 2) == 0, 128: (s & 4) == 0}
    upper_t = {1: (s & 1) != 0, 2: (s & 2) != 0, 4: (s & 4) != 0}

    def cross(jv, k):
        kb = k.bit_length() - 1
        for a in range(_NV):
            if a & jv:
                continue
            b = a ^ jv
            mn = jnp.minimum(v[a], v[b])
            mx = jnp.maximum(v[a], v[b])
            if kb >= 8:
                v[a], v[b] = mn, mx
            elif kb <= 4:
                if (a & k) == 0:
                    v[a], v[b] = mn, mx
                else:
                    v[a], v[b] = mx, mn
            else:
                m = asc_s[k]
                v[a] = jnp.where(m, mn, mx)
                v[b] = jnp.where(m, mx, mn)

    def sublane(t, k):
        kb = k.bit_length() - 1
        up = upper_t[t]
        if kb >= 8:
            tm = jnp.logical_not(up)
        else:
            tm = up != asc_s[k]
        for a in range(_NV):
            y = v[a]
            if t == 4:
                partner = pltpu.roll(y, 4, axis=0)
            else:
                partner = jnp.where(up,
                                    pltpu.roll(y, t, axis=0),
                                    pltpu.roll(y, 8 - t, axis=0))
            v[a] = jnp.where(tm, jnp.minimum(y, partner),
                             jnp.maximum(y, partner))

    k = 2
    while k <= _N:
        j = k // 2
        while j >= 1:
            b = j.bit_length() - 1
            if b <= 4:
                cross(1 << b, k)
            else:
                sublane(1 << (b - 5), k)
            j //= 2
        k *= 2

    tot = v[0]
    for a in range(1, _NV):
        tot = tot + v[a]
    keep = (s >= 1) & (s <= 6)
    tsum = jnp.sum(jnp.where(keep, tot, 0.0), axis=0, keepdims=True)
    u = tsum / (_N - 2 * _TRIM)
    o_ref[0:1, pl.ds(base, _W)] = jnp.log(jnp.maximum(u, _EPS))


_CHUNKS = 8
_WB = _W * _CHUNKS


def _trim_body(x_ref, o_ref):
    def body(c, carry):
        _chunk(x_ref, o_ref, c)
        return carry

    jax.lax.fori_loop(0, _CHUNKS, body, 0)


@jax.jit
def kernel(x):
    n, d = x.shape
    grid = pl.cdiv(d, _WB)
    out = pl.pallas_call(
        _trim_body,
        grid=(grid,),
        in_specs=[pl.BlockSpec((n, _WB), lambda i: (0, i))],
        out_specs=pl.BlockSpec((1, _WB), lambda i: (0, i)),
        out_shape=jax.ShapeDtypeStruct((1, d), x.dtype),
        compiler_params=pltpu.CompilerParams(
            dimension_semantics=("parallel",),
        ),
    )(x)
    return out.reshape(d)


# speedup vs baseline: 14.7906x; 1.1340x over previous
import jax
import jax.numpy as jnp
from jax.experimental import pallas as pl
from jax.experimental.pallas import tpu as pltpu

_TRIM = 32
_N = 256
_EPS = 1e-12
_W = 128
_NV = _N // 8


def _chunk(x_ref, o_ref, c):
    base = c * _W
    v = [x_ref[i * 8:(i + 1) * 8, pl.ds(base, _W)] for i in range(_NV)]

    s = jax.lax.broadcasted_iota(jnp.int32, (8, 1), 0)
    upper_t = {1: (s & 1) != 0, 2: (s & 2) != 0, 4: (s & 4) != 0}

    def cross(jv, k):
        kb = k.bit_length() - 1
        for a in range(_NV):
            if a & jv:
                continue
            b = a ^ jv
            mn = jnp.minimum(v[a], v[b])
            mx = jnp.maximum(v[a], v[b])
            if kb <= 4 and (a & k) != 0:
                v[a], v[b] = mx, mn
            else:
                v[a], v[b] = mn, mx

    def sublane(t):
        up = upper_t[t]
        for a in range(_NV):
            y = v[a]
            if t == 4:
                partner = pltpu.roll(y, 4, axis=0)
            else:
                partner = jnp.where(up,
                                    pltpu.roll(y, t, axis=0),
                                    pltpu.roll(y, 8 - t, axis=0))
            v[a] = jnp.where(up, jnp.maximum(y, partner),
                             jnp.minimum(y, partner))

    def flip(mask):
        sgn = jnp.where(mask, -1.0, 1.0).astype(jnp.float32)
        for a in range(_NV):
            v[a] = v[a] * sgn

    flips = {32: (s & 1) != 0,
             64: ((s & 1) != 0) != ((s & 2) != 0),
             128: ((s & 2) != 0) != ((s & 4) != 0),
             256: (s & 4) != 0}

    k = 2
    while k <= _N:
        if k in flips:
            flip(flips[k])
        j = k // 2
        while j >= 1:
            b = j.bit_length() - 1
            if b <= 4:
                cross(1 << b, k)
            else:
                sublane(1 << (b - 5))
            j //= 2
        k *= 2

    def tree_sum(lst):
        while len(lst) > 1:
            lst = [lst[i] + lst[i + 1] for i in range(0, len(lst) - 1, 2)] + (
                [lst[-1]] if len(lst) % 2 else [])
        return lst[0]

    tot = tree_sum(list(v))
    keep = (s >= 1) & (s <= 6)
    tsum = jnp.sum(jnp.where(keep, tot, 0.0), axis=0, keepdims=True)
    u = tsum / (_N - 2 * _TRIM)
    o_ref[0:1, pl.ds(base, _W)] = jnp.log(jnp.maximum(u, _EPS))


_CHUNKS = 8
_WB = _W * _CHUNKS


def _trim_body(x_ref, o_ref):
    def body(c, carry):
        _chunk(x_ref, o_ref, c)
        return carry

    jax.lax.fori_loop(0, _CHUNKS, body, 0)


@jax.jit
def kernel(x):
    n, d = x.shape
    grid = pl.cdiv(d, _WB)
    out = pl.pallas_call(
        _trim_body,
        grid=(grid,),
        in_specs=[pl.BlockSpec((n, _WB), lambda i: (0, i))],
        out_specs=pl.BlockSpec((1, _WB), lambda i: (0, i)),
        out_shape=jax.ShapeDtypeStruct((1, d), x.dtype),
        compiler_params=pltpu.CompilerParams(
            dimension_semantics=("parallel",),
        ),
    )(x)
    return out.reshape(d)


# speedup vs baseline: 16.1832x; 1.0941x over previous
import jax
import jax.numpy as jnp
from jax.experimental import pallas as pl
from jax.experimental.pallas import tpu as pltpu

_TRIM = 32
_N = 256
_EPS = 1e-12
_W = 128
_NV = _N // 8


def _chunk(x_ref, o_ref, c):
    base = c * _W
    v = [x_ref[i * 8:(i + 1) * 8, pl.ds(base, _W)] for i in range(_NV)]

    s = jax.lax.broadcasted_iota(jnp.int32, (8, 1), 0)
    upper_t = {1: (s & 1) != 0, 2: (s & 2) != 0, 4: (s & 4) != 0}

    def cross(jv, k):
        kb = k.bit_length() - 1
        for a in range(_NV):
            if a & jv:
                continue
            b = a ^ jv
            mn = jnp.minimum(v[a], v[b])
            mx = jnp.maximum(v[a], v[b])
            if kb <= 4 and (a & k) != 0:
                v[a], v[b] = mx, mn
            else:
                v[a], v[b] = mn, mx

    def sublane(t):
        up = upper_t[t]
        for a in range(_NV):
            y = v[a]
            if t == 4:
                partner = pltpu.roll(y, 4, axis=0)
            else:
                partner = jnp.where(up,
                                    pltpu.roll(y, t, axis=0),
                                    pltpu.roll(y, 8 - t, axis=0))
            v[a] = jnp.where(up, jnp.maximum(y, partner),
                             jnp.minimum(y, partner))

    def flip(mask):
        sgn = jnp.where(mask, -1.0, 1.0).astype(jnp.float32)
        for a in range(_NV):
            v[a] = v[a] * sgn

    strides = {5: 4, 6: 1, 7: 2}
    f4 = (s & 4) != 0
    f1 = (s & 1) != 0
    f2 = (s & 2) != 0
    flips = {32: f4,
             64: f4 != f1,
             128: f1 != f2,
             256: f2}

    k = 2
    while k <= _N:
        if k in flips:
            flip(flips[k])
        j = k // 2
        while j >= 1:
            b = j.bit_length() - 1
            if b <= 4:
                if k < _N:
                    cross(1 << b, k)
            else:
                sublane(strides[b])
            j //= 2
        k *= 2

    def tree_sum(lst):
        while len(lst) > 1:
            lst = [lst[i] + lst[i + 1] for i in range(0, len(lst) - 1, 2)] + (
                [lst[-1]] if len(lst) % 2 else [])
        return lst[0]

    tot = tree_sum(list(v))
    keep = (s >= 1) & (s <= 6)
    tsum = jnp.sum(jnp.where(keep, tot, 0.0), axis=0, keepdims=True)
    u = tsum / (_N - 2 * _TRIM)
    o_ref[0:1, pl.ds(base, _W)] = jnp.log(jnp.maximum(u, _EPS))


_CHUNKS = 8
_WB = _W * _CHUNKS


def _trim_body(x_ref, o_ref):
    def body(c, carry):
        _chunk(x_ref, o_ref, c)
        return carry

    jax.lax.fori_loop(0, _CHUNKS, body, 0)


@jax.jit
def kernel(x):
    n, d = x.shape
    grid = pl.cdiv(d, _WB)
    out = pl.pallas_call(
        _trim_body,
        grid=(grid,),
        in_specs=[pl.BlockSpec((n, _WB), lambda i: (0, i))],
        out_specs=pl.BlockSpec((1, _WB), lambda i: (0, i)),
        out_shape=jax.ShapeDtypeStruct((1, d), x.dtype),
        compiler_params=pltpu.CompilerParams(
            dimension_semantics=("parallel",),
        ),
    )(x)
    return out.reshape(d)


# speedup vs baseline: 16.2119x; 1.0018x over previous
import jax
import jax.numpy as jnp
from jax.experimental import pallas as pl
from jax.experimental.pallas import tpu as pltpu

_TRIM = 32
_N = 256
_EPS = 1e-12
_W = 128
_NV = _N // 8


def _chunk(x_ref, o_ref, c):
    base = c * _W
    v = [x_ref[i * 8:(i + 1) * 8, pl.ds(base, _W)] for i in range(_NV)]

    s = jax.lax.broadcasted_iota(jnp.int32, (8, 1), 0)
    upper_t = {1: (s & 1) != 0, 2: (s & 2) != 0, 4: (s & 4) != 0}

    def cross(jv, k):
        kb = k.bit_length() - 1
        for a in range(_NV):
            if a & jv:
                continue
            b = a ^ jv
            mn = jnp.minimum(v[a], v[b])
            mx = jnp.maximum(v[a], v[b])
            if kb <= 4 and (a & k) != 0:
                v[a], v[b] = mx, mn
            else:
                v[a], v[b] = mn, mx

    def sublane(t):
        up = upper_t[t]
        for a in range(_NV):
            y = v[a]
            if t == 4:
                partner = pltpu.roll(y, 4, axis=0)
            else:
                partner = jnp.where(up,
                                    pltpu.roll(y, t, axis=0),
                                    pltpu.roll(y, 8 - t, axis=0))
            v[a] = jnp.where(up, jnp.maximum(y, partner),
                             jnp.minimum(y, partner))

    def flip(mask):
        sgn = jnp.where(mask, -1.0, 1.0).astype(jnp.float32)
        for a in range(_NV):
            v[a] = v[a] * sgn

    strides = {5: 4, 6: 1, 7: 2}
    f4 = (s & 4) != 0
    f1 = (s & 1) != 0
    f2 = (s & 2) != 0
    flips = {32: f4,
             64: f4 != f1,
             128: f1 != f2,
             256: f2}

    k = 2
    while k <= _N:
        if k in flips:
            flip(flips[k])
        j = k // 2
        while j >= 1:
            b = j.bit_length() - 1
            if b <= 4:
                if k < _N:
                    cross(1 << b, k)
            else:
                sublane(strides[b])
            j //= 2
        k *= 2

    def tree_sum(lst):
        while len(lst) > 1:
            lst = [lst[i] + lst[i + 1] for i in range(0, len(lst) - 1, 2)] + (
                [lst[-1]] if len(lst) % 2 else [])
        return lst[0]

    tot = tree_sum(list(v))
    keep = (s >= 1) & (s <= 6)
    tsum = jnp.sum(jnp.where(keep, tot, 0.0), axis=0, keepdims=True)
    u = tsum / (_N - 2 * _TRIM)
    o_ref[0:1, pl.ds(base, _W)] = jnp.log(jnp.maximum(u, _EPS))


_CHUNKS = 16
_WB = _W * _CHUNKS


def _trim_body(x_ref, o_ref):
    def body(c, carry):
        _chunk(x_ref, o_ref, c)
        return carry

    jax.lax.fori_loop(0, _CHUNKS, body, 0)


@jax.jit
def kernel(x):
    n, d = x.shape
    grid = pl.cdiv(d, _WB)
    out = pl.pallas_call(
        _trim_body,
        grid=(grid,),
        in_specs=[pl.BlockSpec((n, _WB), lambda i: (0, i))],
        out_specs=pl.BlockSpec((1, _WB), lambda i: (0, i)),
        out_shape=jax.ShapeDtypeStruct((1, d), x.dtype),
        compiler_params=pltpu.CompilerParams(
            dimension_semantics=("parallel",),
        ),
    )(x)
    return out.reshape(d)
